# baseline (speedup 1.0000x reference)
import functools

import jax
import jax.numpy as jnp
from jax import lax
from jax.experimental import pallas as pl
from jax.experimental.pallas import tpu as pltpu
from jax.experimental.pallas import tpu_sc as plsc

HIDDEN = 128
SEQ = 33
EPS = 1e-12
N = 10000

NC = 2
NS = 16
NW = NC * NS
L = 16
NV = HIDDEN // L

C = 4
NCH = N // C
D = 3
RG = 4


def _rsqrt(x):
    i = lax.bitcast_convert_type(x, jnp.int32)
    i = jnp.int32(0x5F3759DF) - lax.shift_right_logical(i, jnp.full((L,), 1, jnp.int32))
    y = lax.bitcast_convert_type(i, jnp.float32)
    for _ in range(3):
        y = y * (1.5 - 0.5 * x * y * y)
    return y


def _allsum(v):
    for sh in (1, 2, 4, 8):
        idx = lax.iota(jnp.int32, L) ^ sh
        v = v + v.at[idx].get(mode="promise_in_bounds", unique_indices=True)
    return v


_MESH = plsc.VectorSubcoreMesh(
    core_axis_name="c", subcore_axis_name="s", num_cores=NC, num_subcores=NS)


@functools.partial(
    pl.kernel,
    out_type=jax.ShapeDtypeStruct((N, SEQ, HIDDEN), jnp.float32),
    mesh=_MESH,
    scratch_types=[
        pltpu.VMEM((D, C, SEQ, HIDDEN), jnp.float32),
        pltpu.VMEM((D, C, SEQ, HIDDEN), jnp.float32),
        pltpu.VMEM((4, HIDDEN), jnp.float32),
        pltpu.SemaphoreType.DMA((D,)),
        pltpu.SemaphoreType.DMA((D,)),
    ],
)
def _sc_ln(x_hbm, tt_hbm, w_hbm, b_hbm, o_hbm, in_buf, out_buf, par, in_sem, out_sem):
    wid = lax.axis_index("s") * NC + lax.axis_index("c")

    pltpu.sync_copy(tt_hbm, par.at[0:2])
    pltpu.sync_copy(w_hbm, par.at[2])
    pltpu.sync_copy(b_hbm, par.at[3])

    t0 = [par[0, pl.ds(j * L, L)] for j in range(NV)]
    t1 = [par[1, pl.ds(j * L, L)] for j in range(NV)]
    lw = [par[2, pl.ds(j * L, L)] for j in range(NV)]
    lb = [par[3, pl.ds(j * L, L)] for j in range(NV)]

    nk = (NCH - wid + NW - 1) // NW

    def in_copy(k, s):
        c = wid + k * NW
        return pltpu.make_async_copy(
            x_hbm.at[pl.ds(c * C, C)], in_buf.at[s], in_sem.at[s])

    def out_copy(k, s):
        c = wid + k * NW
        return pltpu.make_async_copy(
            out_buf.at[s], o_hbm.at[pl.ds(c * C, C)], out_sem.at[s])

    def ln_row(slot, n, r, tt):
        acc = None
        acc2 = None
        for j in range(NV):
            v = in_buf[slot, n, r, pl.ds(j * L, L)] + tt[j]
            acc = v if acc is None else acc + v
            acc2 = v * v if acc2 is None else acc2 + v * v
        mean = _allsum(acc) * (1.0 / HIDDEN)
        var = _allsum(acc2) * (1.0 / HIDDEN) - mean * mean
        a = _rsqrt(var + EPS)
        for j in range(NV):
            v = in_buf[slot, n, r, pl.ds(j * L, L)] + tt[j]
            out_buf[slot, n, r, pl.ds(j * L, L)] = (v - mean) * a * lw[j] + lb[j]

    for d in range(D):
        in_copy(d, d).start()

    def step(k, carry):
        slot = k % D
        in_copy(k, slot).wait()

        @pl.when(k >= D)
        def _():
            out_copy(k - D, slot).wait()

        for n in range(C):
            ln_row(slot, n, 0, t0)

            def rgrp(g, cc):
                for rr in range(RG):
                    ln_row(slot, n, 1 + g * RG + rr, t1)
                return cc

            lax.fori_loop(0, (SEQ - 1) // RG, rgrp, 0)

        out_copy(k, slot).start()

        @pl.when(k + D < nk)
        def _():
            in_copy(k + D, slot).start()

        return carry

    lax.fori_loop(0, nk, step, 0)

    for j in range(D):
        kk = nk - D + j
        out_copy(kk, kk % D).wait()


def kernel(input_embed, token_type_table, ln_weight, ln_bias):
    return _sc_ln(input_embed, token_type_table, ln_weight, ln_bias)


# speedup vs baseline: 1.7726x; 1.7726x over previous
import functools

import jax
import jax.numpy as jnp
from jax import lax
from jax.experimental import pallas as pl
from jax.experimental.pallas import tpu as pltpu
from jax.experimental.pallas import tpu_sc as plsc

HIDDEN = 128
SEQ = 33
EPS = 1e-12
N = 10000

NC = 2
NS = 16
NW = NC * NS
L = 16
NV = HIDDEN // L

C = 4
NCH = N // C
D = 3
RG = 4


def _rsqrt(x):
    i = lax.bitcast_convert_type(x, jnp.int32)
    i = jnp.int32(0x5F3759DF) - lax.shift_right_logical(i, jnp.full((L,), 1, jnp.int32))
    y = lax.bitcast_convert_type(i, jnp.float32)
    for _ in range(3):
        y = y * (1.5 - 0.5 * x * y * y)
    return y


def _allsum(v):
    for sh in (1, 2, 4, 8):
        idx = lax.iota(jnp.int32, L) ^ sh
        v = v + v.at[idx].get(mode="promise_in_bounds", unique_indices=True)
    return v


_MESH = plsc.VectorSubcoreMesh(
    core_axis_name="c", subcore_axis_name="s", num_cores=NC, num_subcores=NS)


@functools.partial(
    pl.kernel,
    out_type=jax.ShapeDtypeStruct((N, SEQ, HIDDEN), jnp.float32),
    mesh=_MESH,
    scratch_types=[
        pltpu.VMEM((D, C, SEQ, HIDDEN), jnp.float32),
        pltpu.VMEM((D, C, SEQ, HIDDEN), jnp.float32),
        pltpu.VMEM((4, HIDDEN), jnp.float32),
        pltpu.SemaphoreType.DMA((D,)),
        pltpu.SemaphoreType.DMA((D,)),
    ],
)
def _sc_ln(x_hbm, tt_hbm, w_hbm, b_hbm, o_hbm, in_buf, out_buf, par, in_sem, out_sem):
    wid = lax.axis_index("s") * NC + lax.axis_index("c")

    pltpu.sync_copy(tt_hbm, par.at[0:2])
    pltpu.sync_copy(w_hbm, par.at[2])
    pltpu.sync_copy(b_hbm, par.at[3])

    t0 = [par[0, pl.ds(j * L, L)] for j in range(NV)]
    t1 = [par[1, pl.ds(j * L, L)] for j in range(NV)]
    lw = [par[2, pl.ds(j * L, L)] for j in range(NV)]
    lb = [par[3, pl.ds(j * L, L)] for j in range(NV)]

    nk = (NCH - wid + NW - 1) // NW

    def in_copy(k, s):
        c = wid + k * NW
        return pltpu.make_async_copy(
            x_hbm.at[pl.ds(c * C, C)], in_buf.at[s], in_sem.at[s])

    def out_copy(k, s):
        c = wid + k * NW
        return pltpu.make_async_copy(
            out_buf.at[s], o_hbm.at[pl.ds(c * C, C)], out_sem.at[s])

    def ln_row(slot, n, r, tt):
        vs = []
        acc = None
        acc2 = None
        for j in range(NV):
            v = in_buf[slot, n, r, pl.ds(j * L, L)] + tt[j]
            vs.append(v)
            acc = v if acc is None else acc + v
            acc2 = v * v if acc2 is None else acc2 + v * v
        mean = _allsum(acc) * (1.0 / HIDDEN)
        var = _allsum(acc2) * (1.0 / HIDDEN) - mean * mean
        a = _rsqrt(var + EPS)
        for j in range(NV):
            out_buf[slot, n, r, pl.ds(j * L, L)] = (vs[j] - mean) * a * lw[j] + lb[j]

    for d in range(D):
        in_copy(d, d).start()

    def step(k, carry):
        slot = k % D
        in_copy(k, slot).wait()

        @pl.when(k >= D)
        def _():
            out_copy(k - D, slot).wait()

        for n in range(C):
            ln_row(slot, n, 0, t0)

            @plsc.parallel_loop(1, SEQ, unroll=RG)
            def _(r):
                ln_row(slot, n, r, t1)

        out_copy(k, slot).start()

        @pl.when(k + D < nk)
        def _():
            in_copy(k + D, slot).start()

        return carry

    lax.fori_loop(0, nk, step, 0)

    for j in range(D):
        kk = nk - D + j
        out_copy(kk, kk % D).wait()


def kernel(input_embed, token_type_table, ln_weight, ln_bias):
    return _sc_ln(input_embed, token_type_table, ln_weight, ln_bias)


# speedup vs baseline: 2.0005x; 1.1286x over previous
import functools

import jax
import jax.numpy as jnp
from jax import lax
from jax.experimental import pallas as pl
from jax.experimental.pallas import tpu as pltpu
from jax.experimental.pallas import tpu_sc as plsc

HIDDEN = 128
SEQ = 33
EPS = 1e-12
N = 10000

NC = 2
NS = 16
NW = NC * NS
L = 16
NV = HIDDEN // L

C = 4
NCH = N // C
D = 3
RG = 4


def _rsqrt(x):
    i = lax.bitcast_convert_type(x, jnp.int32)
    i = jnp.int32(0x5F3759DF) - lax.shift_right_logical(i, jnp.full((L,), 1, jnp.int32))
    y = lax.bitcast_convert_type(i, jnp.float32)
    for _ in range(2):
        y = y * (1.5 - 0.5 * x * y * y)
    return y


def _allsum(v):
    for sh in (1, 2, 4, 8):
        idx = lax.iota(jnp.int32, L) ^ sh
        v = v + v.at[idx].get(mode="promise_in_bounds", unique_indices=True)
    return v


_MESH = plsc.VectorSubcoreMesh(
    core_axis_name="c", subcore_axis_name="s", num_cores=NC, num_subcores=NS)


@functools.partial(
    pl.kernel,
    out_type=jax.ShapeDtypeStruct((N, SEQ, HIDDEN), jnp.float32),
    mesh=_MESH,
    scratch_types=[
        pltpu.VMEM((D, C, SEQ, HIDDEN), jnp.float32),
        pltpu.VMEM((D, C, SEQ, HIDDEN), jnp.float32),
        pltpu.VMEM((4, HIDDEN), jnp.float32),
        pltpu.SemaphoreType.DMA((D,)),
        pltpu.SemaphoreType.DMA((D,)),
    ],
)
def _sc_ln(x_hbm, tt_hbm, w_hbm, b_hbm, o_hbm, in_buf, out_buf, par, in_sem, out_sem):
    wid = lax.axis_index("s") * NC + lax.axis_index("c")

    pltpu.sync_copy(tt_hbm, par.at[0:2])
    pltpu.sync_copy(w_hbm, par.at[2])
    pltpu.sync_copy(b_hbm, par.at[3])

    t0 = [par[0, pl.ds(j * L, L)] for j in range(NV)]
    t1 = [par[1, pl.ds(j * L, L)] for j in range(NV)]
    lw = [par[2, pl.ds(j * L, L)] for j in range(NV)]
    lb = [par[3, pl.ds(j * L, L)] for j in range(NV)]

    nk = (NCH - wid + NW - 1) // NW

    def in_copy(k, s):
        c = wid + k * NW
        return pltpu.make_async_copy(
            x_hbm.at[pl.ds(c * C, C)], in_buf.at[s], in_sem.at[s])

    def out_copy(k, s):
        c = wid + k * NW
        return pltpu.make_async_copy(
            out_buf.at[s], o_hbm.at[pl.ds(c * C, C)], out_sem.at[s])

    def ln_row(slot, n, r, tt):
        vs = []
        acc = None
        acc2 = None
        for j in range(NV):
            v = in_buf[slot, n, r, pl.ds(j * L, L)] + tt[j]
            vs.append(v)
            acc = v if acc is None else acc + v
            acc2 = v * v if acc2 is None else acc2 + v * v
        mean = _allsum(acc) * (1.0 / HIDDEN)
        var = _allsum(acc2) * (1.0 / HIDDEN) - mean * mean
        a = _rsqrt(var + EPS)
        for j in range(NV):
            out_buf[slot, n, r, pl.ds(j * L, L)] = (vs[j] - mean) * a * lw[j] + lb[j]

    for d in range(D):
        in_copy(d, d).start()

    def step(k, carry):
        slot = k % D
        in_copy(k, slot).wait()

        @pl.when(k >= D)
        def _():
            out_copy(k - D, slot).wait()

        for n in range(C):
            ln_row(slot, n, 0, t0)

            @plsc.parallel_loop(1, SEQ, unroll=RG)
            def _(r):
                ln_row(slot, n, r, t1)

        out_copy(k, slot).start()

        @pl.when(k + D < nk)
        def _():
            in_copy(k + D, slot).start()

        return carry

    lax.fori_loop(0, nk, step, 0)

    for j in range(D):
        kk = nk - D + j
        out_copy(kk, kk % D).wait()


def kernel(input_embed, token_type_table, ln_weight, ln_bias):
    return _sc_ln(input_embed, token_type_table, ln_weight, ln_bias)
